# baseline (speedup 1.0000x reference)
import functools

import jax
import jax.numpy as jnp
from jax import lax
from jax.experimental import pallas as pl
from jax.experimental.pallas import tpu as pltpu
from jax.experimental.pallas import tpu_sc as plsc

N = 10000
D = 128
E = 320000
DOUT = 64

NUM_SC = 2
NUM_TILES = 16
LANES = 16
DH = D // NUM_SC
CHUNK = 256
CNT_BLK = 128
CHUNKS_PER_TILE = 80
QUARTER = CHUNKS_PER_TILE // 4
EDGES_PER_TYPE = CHUNK * CHUNKS_PER_TILE * NUM_TILES
N_CHUNKS = EDGES_PER_TYPE // CHUNK
NBUF = 4
GAHEAD = 2
NCBUF = 4

NP = 10112
ROWS_PER_TILE = NP // NUM_TILES

ROW_BLOCK = 1000


def _sc_segsum_builder(with_counts: bool):
  mesh = plsc.VectorSubcoreMesh(core_axis_name="c", subcore_axis_name="s",
                                num_cores=NUM_SC, num_subcores=NUM_TILES)

  out_type = [jax.ShapeDtypeStruct((NUM_SC, 2, NP, DH), jnp.float32)]
  if with_counts:
    out_type.append(jax.ShapeDtypeStruct((2, NP, LANES), jnp.float32))

  scratch = [
      pltpu.VMEM((QUARTER, CHUNK), jnp.int32),
      pltpu.VMEM((QUARTER, CHUNK), jnp.int32),
      pltpu.VMEM((NBUF, CHUNK, DH), jnp.float32),
      pltpu.VMEM_SHARED((NP, DH), jnp.float32),
  ] + [pltpu.SemaphoreType.DMA] * (2 * NBUF)
  if with_counts:
    scratch += [
        pltpu.VMEM((CNT_BLK, LANES), jnp.float32),
        pltpu.VMEM_SHARED((NP, LANES), jnp.float32),
    ] + [pltpu.SemaphoreType.DMA] * NCBUF

  def body(hsl, edges, sums, *rest):
    if with_counts:
      (cnt_out, src_i, dst_i, rows, accum) = rest[:5]
      gsems = rest[5:5 + NBUF]
      ssems = rest[5 + NBUF:5 + 2 * NBUF]
      ob, cntacc = rest[5 + 2 * NBUF:7 + 2 * NBUF]
      csems = rest[7 + 2 * NBUF:]
    else:
      (src_i, dst_i, rows, accum) = rest[:4]
      gsems = rest[4:4 + NBUF]
      ssems = rest[4 + NBUF:4 + 2 * NBUF]
      cnt_out = ob = cntacc = csems = None

    c = lax.axis_index("c")
    w = lax.axis_index("s")
    base_row = w * ROWS_PER_TILE
    stripe = pl.ds(base_row, ROWS_PER_TILE)

    zero16 = jnp.zeros((LANES,), jnp.float32)

    nfull = ROWS_PER_TILE // CHUNK
    rem = ROWS_PER_TILE - nfull * CHUNK
    nfullc = ROWS_PER_TILE // CNT_BLK
    remc = ROWS_PER_TILE - nfullc * CNT_BLK

    for p in range(2):
      @pl.loop(0, CHUNK)
      def _(r):
        @pl.loop(0, DH // LANES)
        def _(k):
          rows[0, r, pl.ds(k * LANES, LANES)] = zero16

      @pl.loop(0, nfull)
      def _(k):
        pltpu.sync_copy(rows.at[0],
                        accum.at[pl.ds(base_row + k * CHUNK, CHUNK)])
      pltpu.sync_copy(rows.at[0].at[pl.ds(0, rem)],
                      accum.at[pl.ds(base_row + nfull * CHUNK, rem)])

      if with_counts:
        @pl.when(c == p)
        def _():
          @pl.loop(0, CNT_BLK)
          def _(r):
            ob[r, :] = zero16

          @pl.loop(0, nfullc)
          def _(k):
            pltpu.sync_copy(ob,
                            cntacc.at[pl.ds(base_row + k * CNT_BLK, CNT_BLK)])
          pltpu.sync_copy(ob.at[pl.ds(0, remc)],
                          cntacc.at[pl.ds(base_row + nfullc * CNT_BLK, remc)])

          one16 = jnp.full((LANES,), 1.0, jnp.float32)

          @pl.loop(0, CNT_BLK)
          def _(r):
            ob[r, :] = one16

      plsc.subcore_barrier()

      h_src = hsl.at[c, p]

      def gather_desc(j, b):
        return pltpu.make_async_copy(h_src.at[src_i.at[j]], rows.at[b],
                                     gsems[b])

      def scatter_start(j, b):
        pltpu.async_copy(rows.at[b], accum.at[dst_i.at[j]], ssems[b],
                         add=True)

      def scatter_wait(j, b):
        pltpu.make_async_copy(rows.at[b], accum.at[dst_i.at[j]],
                              ssems[b]).wait()

      def count_args(j, k, sl):
        idx = dst_i.at[j].at[pl.ds(k * CNT_BLK, CNT_BLK)]
        return ob, cntacc.at[idx], csems[sl]

      for q in range(4):
        cs = pl.ds(w * CHUNKS_PER_TILE + q * QUARTER, QUARTER)
        pltpu.sync_copy(edges.at[p, 0, cs], src_i)
        pltpu.sync_copy(edges.at[p, 1, cs], dst_i)

        for b in range(GAHEAD):
          gather_desc(b, b).start()

        @pl.loop(0, QUARTER, step=NBUF)
        def _(j):
          for d in range(NBUF):
            jj = j + d
            gather_desc(jj, d).wait()
            scatter_start(jj, d)
            if with_counts:
              @pl.when(c == p)
              def _():
                for k in range(2):
                  sl = (2 * d + k) % NCBUF

                  @pl.when(jj >= 2)
                  def _():
                    pltpu.make_async_copy(*count_args(jj - 2, k, sl)).wait()
                  pltpu.async_copy(*count_args(jj, k, sl), add=True)
            nxt = jj + GAHEAD
            nb = (d + GAHEAD) % NBUF

            @pl.when(nxt < QUARTER)
            def _():
              @pl.when(nxt >= NBUF)
              def _():
                scatter_wait(nxt - NBUF, nb)
              gather_desc(nxt, nb).start()

        for d in range(NBUF):
          scatter_wait(QUARTER - NBUF + d, d)
        if with_counts:
          @pl.when(c == p)
          def _():
            for x in range(QUARTER - 2, QUARTER):
              for k in range(2):
                pltpu.make_async_copy(
                    *count_args(x, k, (2 * x + k) % NCBUF)).wait()

      plsc.subcore_barrier()

      pltpu.sync_copy(accum.at[stripe], sums.at[c, p].at[stripe])
      if with_counts:
        @pl.when(c == p)
        def _():
          pltpu.sync_copy(cntacc.at[stripe], cnt_out.at[p].at[stripe])

  return pl.kernel(body, out_type=out_type, mesh=mesh, scratch_types=scratch,
                   compiler_params=pltpu.CompilerParams(use_tc_tiling_on_sc=False),
                   name="sc_segsum_cnt" if with_counts else "sc_segsum")


_sc_segsum_builder = functools.cache(_sc_segsum_builder)


def _ln_relu(y, g, bb):
  mu = jnp.mean(y, axis=1, keepdims=True)
  var = jnp.mean((y - mu) ** 2, axis=1, keepdims=True)
  z = (y - mu) * lax.rsqrt(var + 1e-5) * g + bb
  return jnp.maximum(z, 0.0)


def _sage(sums, hs, cnt, wl, wr, b):
  count = jnp.maximum(cnt[0, :, 0:1], 1.0)
  mean = jnp.concatenate([sums[0, 0], sums[1, 0]], axis=-1) / count
  return (jnp.dot(mean, wl[0], preferred_element_type=jnp.float32)
          + jnp.dot(hs[0], wr[0], preferred_element_type=jnp.float32)
          + b[0])


def _tc_layer_body(sums, hs, cnt, wl, wr, b, g, bb, out):
  y = _sage(sums, hs, cnt, wl, wr, b)
  out[0] = _ln_relu(y, g[0], bb[0])


def _tc_layer_head_body(sums, hs, cnt, wl, wr, b, g, bb, wh, bh, out):
  y = _sage(sums, hs, cnt, wl, wr, b)
  h = _ln_relu(y, g[0], bb[0])
  out[0] = jnp.dot(h, wh[0], preferred_element_type=jnp.float32) + bh[0]


def _sums_spec():
  return pl.BlockSpec((NUM_SC, 1, ROW_BLOCK, DH), lambda t, k: (0, 1 - t, k, 0))


def _row_spec(width, flip):
  if flip:
    return pl.BlockSpec((1, ROW_BLOCK, width), lambda t, k: (1 - t, k, 0))
  return pl.BlockSpec((1, ROW_BLOCK, width), lambda t, k: (t, k, 0))


def _w_spec(d1, d2):
  return pl.BlockSpec((1, d1, d2), lambda t, k: (t, 0, 0))


def _tc_layer(sums, cnt, hs, wl, wr, b, g, bb):
  return pl.pallas_call(
      _tc_layer_body,
      grid=(2, N // ROW_BLOCK),
      in_specs=[
          _sums_spec(),
          _row_spec(D, False),
          _row_spec(LANES, True),
          _w_spec(D, D), _w_spec(D, D), _w_spec(1, D),
          _w_spec(1, D), _w_spec(1, D),
      ],
      out_specs=_row_spec(D, False),
      out_shape=jax.ShapeDtypeStruct((2, N, D), jnp.float32),
      name="tc_layer",
  )(sums, hs, cnt, wl, wr, b, g, bb)


def _tc_layer_head(sums, cnt, hs, wl, wr, b, g, bb, wh, bh):
  return pl.pallas_call(
      _tc_layer_head_body,
      grid=(2, N // ROW_BLOCK),
      in_specs=[
          _sums_spec(),
          _row_spec(D, False),
          _row_spec(LANES, True),
          _w_spec(D, D), _w_spec(D, D), _w_spec(1, D),
          _w_spec(1, D), _w_spec(1, D),
          _w_spec(D, DOUT), _w_spec(1, DOUT),
      ],
      out_specs=_row_spec(DOUT, False),
      out_shape=jax.ShapeDtypeStruct((2, N, DOUT), jnp.float32),
      name="tc_layer_head",
  )(sums, hs, cnt, wl, wr, b, g, bb, wh, bh)


def _pad_edges(edge_index):
  src = edge_index[0]
  dst = edge_index[1]
  pad = EDGES_PER_TYPE - E
  src = jnp.concatenate([src, jnp.zeros((pad,), jnp.int32)])
  dst = jnp.concatenate([dst, jnp.full((pad,), N, jnp.int32)])
  return jnp.stack([src.reshape(N_CHUNKS, CHUNK), dst.reshape(N_CHUNKS, CHUNK)])


def _slabs(hs):
  return hs.reshape(2, N, NUM_SC, DH).transpose(2, 0, 1, 3)


def _stack_layer_params(lp):
  wl = jnp.stack([lp['i2u']['Wl'], lp['u2i']['Wl']])
  wr = jnp.stack([lp['i2u']['Wr'], lp['u2i']['Wr']])
  b = jnp.stack([lp['i2u']['b'], lp['u2i']['b']])[:, None, :]
  g = jnp.stack([lp['ln_u']['g'], lp['ln_i']['g']])[:, None, :]
  bb = jnp.stack([lp['ln_u']['b'], lp['ln_i']['b']])[:, None, :]
  return wl, wr, b, g, bb


@jax.jit
def kernel(x_user, x_item, edge_index_u2i, edge_index_i2u, params):
  edges = jnp.stack([_pad_edges(edge_index_u2i), _pad_edges(edge_index_i2u)])
  hs = jnp.stack([x_user, x_item])

  lp0, lp1 = params['layers']

  sums, cnt = _sc_segsum_builder(True)(_slabs(hs), edges)
  hs = _tc_layer(sums, cnt, hs, *_stack_layer_params(lp0))

  (sums,) = _sc_segsum_builder(False)(_slabs(hs), edges)
  wh = jnp.stack([params['head']['u']['W'], params['head']['i']['W']])
  bh = jnp.stack([params['head']['u']['b'], params['head']['i']['b']])[:, None, :]
  wl, wr, b, g, bb = _stack_layer_params(lp1)
  out = _tc_layer_head(sums, cnt, hs, wl, wr, b, g, bb, wh, bh)

  return out[0], out[1]


# speedup vs baseline: 1.1304x; 1.1304x over previous
import functools

import jax
import jax.numpy as jnp
from jax import lax
from jax.experimental import pallas as pl
from jax.experimental.pallas import tpu as pltpu
from jax.experimental.pallas import tpu_sc as plsc

N = 10000
D = 128
E = 320000
DOUT = 64

NUM_SC = 2
NUM_TILES = 16
LANES = 16
DH = D // NUM_SC
CHUNK = 128
CNT_BLK = 128
CHUNKS_PER_TILE = 160
SEG = 80
EDGES_PER_TYPE = CHUNK * CHUNKS_PER_TILE * NUM_TILES
N_CHUNKS = EDGES_PER_TYPE // CHUNK
NBUF = 5
GAHEAD = 3
NCBUF = 5

NP = 10112
ROWS_PER_TILE = NP // NUM_TILES

ROW_BLOCK = 1000


def _sc_segsum_builder(with_counts: bool):
  mesh = plsc.VectorSubcoreMesh(core_axis_name="c", subcore_axis_name="s",
                                num_cores=NUM_SC, num_subcores=NUM_TILES)

  out_type = [jax.ShapeDtypeStruct((NUM_SC, 2, NP, DH), jnp.float32)]
  if with_counts:
    out_type.append(jax.ShapeDtypeStruct((2, NP, LANES), jnp.float32))

  scratch = [
      pltpu.VMEM((SEG, CHUNK), jnp.int32),
      pltpu.VMEM((SEG, CHUNK), jnp.int32),
      pltpu.VMEM((NBUF, CHUNK, DH), jnp.float32),
      pltpu.VMEM_SHARED((NP, DH), jnp.float32),
  ] + [pltpu.SemaphoreType.DMA] * (2 * NBUF)
  if with_counts:
    scratch += [
        pltpu.VMEM((CNT_BLK, LANES), jnp.float32),
        pltpu.VMEM_SHARED((NP, LANES), jnp.float32),
    ] + [pltpu.SemaphoreType.DMA] * NCBUF

  def body(hsl, edges, sums, *rest):
    if with_counts:
      (cnt_out, src_i, dst_i, rows, accum) = rest[:5]
      gsems = rest[5:5 + NBUF]
      ssems = rest[5 + NBUF:5 + 2 * NBUF]
      ob, cntacc = rest[5 + 2 * NBUF:7 + 2 * NBUF]
      csems = rest[7 + 2 * NBUF:]
    else:
      (src_i, dst_i, rows, accum) = rest[:4]
      gsems = rest[4:4 + NBUF]
      ssems = rest[4 + NBUF:4 + 2 * NBUF]
      cnt_out = ob = cntacc = csems = None

    c = lax.axis_index("c")
    w = lax.axis_index("s")
    base_row = w * ROWS_PER_TILE
    stripe = pl.ds(base_row, ROWS_PER_TILE)

    zero16 = jnp.zeros((LANES,), jnp.float32)

    nfull = ROWS_PER_TILE // CHUNK
    rem = ROWS_PER_TILE - nfull * CHUNK
    nfullc = ROWS_PER_TILE // CNT_BLK
    remc = ROWS_PER_TILE - nfullc * CNT_BLK

    for p in range(2):
      @pl.loop(0, CHUNK)
      def _(r):
        @pl.loop(0, DH // LANES)
        def _(k):
          rows[0, r, pl.ds(k * LANES, LANES)] = zero16

      @pl.loop(0, nfull)
      def _(k):
        pltpu.sync_copy(rows.at[0],
                        accum.at[pl.ds(base_row + k * CHUNK, CHUNK)])
      pltpu.sync_copy(rows.at[0].at[pl.ds(0, rem)],
                      accum.at[pl.ds(base_row + nfull * CHUNK, rem)])

      if with_counts:
        @pl.when(c == p)
        def _():
          @pl.loop(0, CNT_BLK)
          def _(r):
            ob[r, :] = zero16

          @pl.loop(0, nfullc)
          def _(k):
            pltpu.sync_copy(ob,
                            cntacc.at[pl.ds(base_row + k * CNT_BLK, CNT_BLK)])
          pltpu.sync_copy(ob.at[pl.ds(0, remc)],
                          cntacc.at[pl.ds(base_row + nfullc * CNT_BLK, remc)])

          one16 = jnp.full((LANES,), 1.0, jnp.float32)

          @pl.loop(0, CNT_BLK)
          def _(r):
            ob[r, :] = one16

      plsc.subcore_barrier()

      h_src = hsl.at[c, p]

      def gather_desc(j, b):
        return pltpu.make_async_copy(h_src.at[src_i.at[j]], rows.at[b],
                                     gsems[b])

      def scatter_start(j, b):
        pltpu.async_copy(rows.at[b], accum.at[dst_i.at[j]], ssems[b],
                         add=True)

      def scatter_wait(j, b):
        pltpu.make_async_copy(rows.at[b], accum.at[dst_i.at[j]],
                              ssems[b]).wait()

      def count_args(j, sl):
        return ob, cntacc.at[dst_i.at[j]], csems[sl]

      for q in range(CHUNKS_PER_TILE // SEG):
        cs = pl.ds(w * CHUNKS_PER_TILE + q * SEG, SEG)
        pltpu.sync_copy(edges.at[p, 0, cs], src_i)
        pltpu.sync_copy(edges.at[p, 1, cs], dst_i)

        for b in range(GAHEAD):
          gather_desc(b, b).start()

        @pl.loop(0, SEG, step=NBUF)
        def _(j):
          for d in range(NBUF):
            jj = j + d
            gather_desc(jj, d).wait()
            scatter_start(jj, d)
            if with_counts:
              @pl.when(c == p)
              def _():
                @pl.when(jj >= NCBUF)
                def _():
                  pltpu.make_async_copy(*count_args(jj - NCBUF, d)).wait()
                pltpu.async_copy(*count_args(jj, d), add=True)
            nxt = jj + GAHEAD
            nb = (d + GAHEAD) % NBUF

            @pl.when(nxt < SEG)
            def _():
              @pl.when(nxt >= NBUF)
              def _():
                scatter_wait(nxt - NBUF, nb)
              gather_desc(nxt, nb).start()

        for d in range(NBUF):
          scatter_wait(SEG - NBUF + d, d)
        if with_counts:
          @pl.when(c == p)
          def _():
            for d in range(NCBUF):
              pltpu.make_async_copy(
                  *count_args(SEG - NCBUF + d, d)).wait()

      plsc.subcore_barrier()

      pltpu.sync_copy(accum.at[stripe], sums.at[c, p].at[stripe])
      if with_counts:
        @pl.when(c == p)
        def _():
          pltpu.sync_copy(cntacc.at[stripe], cnt_out.at[p].at[stripe])

  return pl.kernel(body, out_type=out_type, mesh=mesh, scratch_types=scratch,
                   compiler_params=pltpu.CompilerParams(use_tc_tiling_on_sc=False),
                   name="sc_segsum_cnt" if with_counts else "sc_segsum")


_sc_segsum_builder = functools.cache(_sc_segsum_builder)


def _ln_relu(y, g, bb):
  mu = jnp.mean(y, axis=1, keepdims=True)
  var = jnp.mean((y - mu) ** 2, axis=1, keepdims=True)
  z = (y - mu) * lax.rsqrt(var + 1e-5) * g + bb
  return jnp.maximum(z, 0.0)


def _sage(sums, hs, cnt, wl, wr, b):
  count = jnp.maximum(cnt[0, :, 0:1], 1.0)
  mean = jnp.concatenate([sums[0, 0], sums[1, 0]], axis=-1) / count
  return (jnp.dot(mean, wl[0], preferred_element_type=jnp.float32)
          + jnp.dot(hs[0], wr[0], preferred_element_type=jnp.float32)
          + b[0])


def _tc_layer_body(sums, hs, cnt, wl, wr, b, g, bb, out):
  y = _sage(sums, hs, cnt, wl, wr, b)
  out[0] = _ln_relu(y, g[0], bb[0])


def _tc_layer_head_body(sums, hs, cnt, wl, wr, b, g, bb, wh, bh, out):
  y = _sage(sums, hs, cnt, wl, wr, b)
  h = _ln_relu(y, g[0], bb[0])
  out[0] = jnp.dot(h, wh[0], preferred_element_type=jnp.float32) + bh[0]


def _sums_spec():
  return pl.BlockSpec((NUM_SC, 1, ROW_BLOCK, DH), lambda t, k: (0, 1 - t, k, 0))


def _row_spec(width, flip):
  if flip:
    return pl.BlockSpec((1, ROW_BLOCK, width), lambda t, k: (1 - t, k, 0))
  return pl.BlockSpec((1, ROW_BLOCK, width), lambda t, k: (t, k, 0))


def _w_spec(d1, d2):
  return pl.BlockSpec((1, d1, d2), lambda t, k: (t, 0, 0))


def _tc_layer(sums, cnt, hs, wl, wr, b, g, bb):
  return pl.pallas_call(
      _tc_layer_body,
      grid=(2, N // ROW_BLOCK),
      in_specs=[
          _sums_spec(),
          _row_spec(D, False),
          _row_spec(LANES, True),
          _w_spec(D, D), _w_spec(D, D), _w_spec(1, D),
          _w_spec(1, D), _w_spec(1, D),
      ],
      out_specs=_row_spec(D, False),
      out_shape=jax.ShapeDtypeStruct((2, N, D), jnp.float32),
      name="tc_layer",
  )(sums, hs, cnt, wl, wr, b, g, bb)


def _tc_layer_head(sums, cnt, hs, wl, wr, b, g, bb, wh, bh):
  return pl.pallas_call(
      _tc_layer_head_body,
      grid=(2, N // ROW_BLOCK),
      in_specs=[
          _sums_spec(),
          _row_spec(D, False),
          _row_spec(LANES, True),
          _w_spec(D, D), _w_spec(D, D), _w_spec(1, D),
          _w_spec(1, D), _w_spec(1, D),
          _w_spec(D, DOUT), _w_spec(1, DOUT),
      ],
      out_specs=_row_spec(DOUT, False),
      out_shape=jax.ShapeDtypeStruct((2, N, DOUT), jnp.float32),
      name="tc_layer_head",
  )(sums, hs, cnt, wl, wr, b, g, bb, wh, bh)


def _pad_edges(edge_index):
  src = edge_index[0]
  dst = edge_index[1]
  pad = EDGES_PER_TYPE - E
  src = jnp.concatenate([src, jnp.zeros((pad,), jnp.int32)])
  dst = jnp.concatenate([dst, jnp.full((pad,), N, jnp.int32)])
  return jnp.stack([src.reshape(N_CHUNKS, CHUNK), dst.reshape(N_CHUNKS, CHUNK)])


def _slabs(hs):
  return hs.reshape(2, N, NUM_SC, DH).transpose(2, 0, 1, 3)


def _stack_layer_params(lp):
  wl = jnp.stack([lp['i2u']['Wl'], lp['u2i']['Wl']])
  wr = jnp.stack([lp['i2u']['Wr'], lp['u2i']['Wr']])
  b = jnp.stack([lp['i2u']['b'], lp['u2i']['b']])[:, None, :]
  g = jnp.stack([lp['ln_u']['g'], lp['ln_i']['g']])[:, None, :]
  bb = jnp.stack([lp['ln_u']['b'], lp['ln_i']['b']])[:, None, :]
  return wl, wr, b, g, bb


@jax.jit
def kernel(x_user, x_item, edge_index_u2i, edge_index_i2u, params):
  edges = jnp.stack([_pad_edges(edge_index_u2i), _pad_edges(edge_index_i2u)])
  hs = jnp.stack([x_user, x_item])

  lp0, lp1 = params['layers']

  sums, cnt = _sc_segsum_builder(True)(_slabs(hs), edges)
  hs = _tc_layer(sums, cnt, hs, *_stack_layer_params(lp0))

  (sums,) = _sc_segsum_builder(False)(_slabs(hs), edges)
  wh = jnp.stack([params['head']['u']['W'], params['head']['i']['W']])
  bh = jnp.stack([params['head']['u']['b'], params['head']['i']['b']])[:, None, :]
  wl, wr, b, g, bb = _stack_layer_params(lp1)
  out = _tc_layer_head(sums, cnt, hs, wl, wr, b, g, bb, wh, bh)

  return out[0], out[1]


# speedup vs baseline: 1.1344x; 1.0035x over previous
import functools

import jax
import jax.numpy as jnp
from jax import lax
from jax.experimental import pallas as pl
from jax.experimental.pallas import tpu as pltpu
from jax.experimental.pallas import tpu_sc as plsc

N = 10000
D = 128
E = 320000
DOUT = 64

NUM_SC = 2
NUM_TILES = 16
LANES = 16
DH = D // NUM_SC
CHUNK = 128
CNT_BLK = 128
CHUNKS_PER_TILE = 160
SEG = 80
EDGES_PER_TYPE = CHUNK * CHUNKS_PER_TILE * NUM_TILES
N_CHUNKS = EDGES_PER_TYPE // CHUNK
NBUF = 5
GAHEAD = 4
NCBUF = 5

NP = 10112
ROWS_PER_TILE = NP // NUM_TILES

ROW_BLOCK = 1000


def _sc_segsum_builder(with_counts: bool):
  mesh = plsc.VectorSubcoreMesh(core_axis_name="c", subcore_axis_name="s",
                                num_cores=NUM_SC, num_subcores=NUM_TILES)

  out_type = [jax.ShapeDtypeStruct((NUM_SC, 2, NP, DH), jnp.float32)]
  if with_counts:
    out_type.append(jax.ShapeDtypeStruct((2, NP, LANES), jnp.float32))

  scratch = [
      pltpu.VMEM((SEG, CHUNK), jnp.int32),
      pltpu.VMEM((SEG, CHUNK), jnp.int32),
      pltpu.VMEM((NBUF, CHUNK, DH), jnp.float32),
      pltpu.VMEM_SHARED((NP, DH), jnp.float32),
  ] + [pltpu.SemaphoreType.DMA] * (2 * NBUF)
  if with_counts:
    scratch += [
        pltpu.VMEM((CNT_BLK, LANES), jnp.float32),
        pltpu.VMEM_SHARED((NP, LANES), jnp.float32),
    ] + [pltpu.SemaphoreType.DMA] * NCBUF

  def body(hsl, edges, sums, *rest):
    if with_counts:
      (cnt_out, src_i, dst_i, rows, accum) = rest[:5]
      gsems = rest[5:5 + NBUF]
      ssems = rest[5 + NBUF:5 + 2 * NBUF]
      ob, cntacc = rest[5 + 2 * NBUF:7 + 2 * NBUF]
      csems = rest[7 + 2 * NBUF:]
    else:
      (src_i, dst_i, rows, accum) = rest[:4]
      gsems = rest[4:4 + NBUF]
      ssems = rest[4 + NBUF:4 + 2 * NBUF]
      cnt_out = ob = cntacc = csems = None

    c = lax.axis_index("c")
    w = lax.axis_index("s")
    base_row = w * ROWS_PER_TILE
    stripe = pl.ds(base_row, ROWS_PER_TILE)

    zero16 = jnp.zeros((LANES,), jnp.float32)

    nfull = ROWS_PER_TILE // CHUNK
    rem = ROWS_PER_TILE - nfull * CHUNK
    nfullc = ROWS_PER_TILE // CNT_BLK
    remc = ROWS_PER_TILE - nfullc * CNT_BLK

    for p in range(2):
      @pl.loop(0, CHUNK)
      def _(r):
        @pl.loop(0, DH // LANES)
        def _(k):
          rows[0, r, pl.ds(k * LANES, LANES)] = zero16

      @pl.loop(0, nfull)
      def _(k):
        pltpu.sync_copy(rows.at[0],
                        accum.at[pl.ds(base_row + k * CHUNK, CHUNK)])
      pltpu.sync_copy(rows.at[0].at[pl.ds(0, rem)],
                      accum.at[pl.ds(base_row + nfull * CHUNK, rem)])

      if with_counts:
        @pl.when(c == p)
        def _():
          @pl.loop(0, CNT_BLK)
          def _(r):
            ob[r, :] = zero16

          @pl.loop(0, nfullc)
          def _(k):
            pltpu.sync_copy(ob,
                            cntacc.at[pl.ds(base_row + k * CNT_BLK, CNT_BLK)])
          pltpu.sync_copy(ob.at[pl.ds(0, remc)],
                          cntacc.at[pl.ds(base_row + nfullc * CNT_BLK, remc)])

          one16 = jnp.full((LANES,), 1.0, jnp.float32)

          @pl.loop(0, CNT_BLK)
          def _(r):
            ob[r, :] = one16

      plsc.subcore_barrier()

      h_src = hsl.at[c, p]

      def gather_desc(j, b):
        return pltpu.make_async_copy(h_src.at[src_i.at[j]], rows.at[b],
                                     gsems[b])

      def scatter_start(j, b):
        pltpu.async_copy(rows.at[b], accum.at[dst_i.at[j]], ssems[b],
                         add=True)

      def scatter_wait(j, b):
        pltpu.make_async_copy(rows.at[b], accum.at[dst_i.at[j]],
                              ssems[b]).wait()

      def count_args(j, sl):
        return ob, cntacc.at[dst_i.at[j]], csems[sl]

      for q in range(CHUNKS_PER_TILE // SEG):
        cs = pl.ds(w * CHUNKS_PER_TILE + q * SEG, SEG)
        pltpu.sync_copy(edges.at[p, 0, cs], src_i)
        pltpu.sync_copy(edges.at[p, 1, cs], dst_i)

        for b in range(GAHEAD):
          gather_desc(b, b).start()

        @pl.loop(0, SEG, step=NBUF)
        def _(j):
          for d in range(NBUF):
            jj = j + d
            gather_desc(jj, d).wait()
            scatter_start(jj, d)
            if with_counts:
              @pl.when(c == p)
              def _():
                @pl.when(jj >= NCBUF)
                def _():
                  pltpu.make_async_copy(*count_args(jj - NCBUF, d)).wait()
                pltpu.async_copy(*count_args(jj, d), add=True)
            nxt = jj + GAHEAD
            nb = (d + GAHEAD) % NBUF

            @pl.when(nxt < SEG)
            def _():
              @pl.when(nxt >= NBUF)
              def _():
                scatter_wait(nxt - NBUF, nb)
              gather_desc(nxt, nb).start()

        for d in range(NBUF):
          scatter_wait(SEG - NBUF + d, d)
        if with_counts:
          @pl.when(c == p)
          def _():
            for d in range(NCBUF):
              pltpu.make_async_copy(
                  *count_args(SEG - NCBUF + d, d)).wait()

      plsc.subcore_barrier()

      pltpu.sync_copy(accum.at[stripe], sums.at[c, p].at[stripe])
      if with_counts:
        @pl.when(c == p)
        def _():
          pltpu.sync_copy(cntacc.at[stripe], cnt_out.at[p].at[stripe])

  return pl.kernel(body, out_type=out_type, mesh=mesh, scratch_types=scratch,
                   compiler_params=pltpu.CompilerParams(use_tc_tiling_on_sc=False),
                   name="sc_segsum_cnt" if with_counts else "sc_segsum")


_sc_segsum_builder = functools.cache(_sc_segsum_builder)


def _ln_relu(y, g, bb):
  mu = jnp.mean(y, axis=1, keepdims=True)
  var = jnp.mean((y - mu) ** 2, axis=1, keepdims=True)
  z = (y - mu) * lax.rsqrt(var + 1e-5) * g + bb
  return jnp.maximum(z, 0.0)


def _sage(sums, hs, cnt, wl, wr, b):
  count = jnp.maximum(cnt[0, :, 0:1], 1.0)
  mean = jnp.concatenate([sums[0, 0], sums[1, 0]], axis=-1) / count
  return (jnp.dot(mean, wl[0], preferred_element_type=jnp.float32)
          + jnp.dot(hs[0], wr[0], preferred_element_type=jnp.float32)
          + b[0])


def _tc_layer_body(sums, hs, cnt, wl, wr, b, g, bb, out):
  y = _sage(sums, hs, cnt, wl, wr, b)
  out[0] = _ln_relu(y, g[0], bb[0])


def _tc_layer_head_body(sums, hs, cnt, wl, wr, b, g, bb, wh, bh, out):
  y = _sage(sums, hs, cnt, wl, wr, b)
  h = _ln_relu(y, g[0], bb[0])
  out[0] = jnp.dot(h, wh[0], preferred_element_type=jnp.float32) + bh[0]


def _sums_spec():
  return pl.BlockSpec((NUM_SC, 1, ROW_BLOCK, DH), lambda t, k: (0, 1 - t, k, 0))


def _row_spec(width, flip):
  if flip:
    return pl.BlockSpec((1, ROW_BLOCK, width), lambda t, k: (1 - t, k, 0))
  return pl.BlockSpec((1, ROW_BLOCK, width), lambda t, k: (t, k, 0))


def _w_spec(d1, d2):
  return pl.BlockSpec((1, d1, d2), lambda t, k: (t, 0, 0))


def _tc_layer(sums, cnt, hs, wl, wr, b, g, bb):
  return pl.pallas_call(
      _tc_layer_body,
      grid=(2, N // ROW_BLOCK),
      in_specs=[
          _sums_spec(),
          _row_spec(D, False),
          _row_spec(LANES, True),
          _w_spec(D, D), _w_spec(D, D), _w_spec(1, D),
          _w_spec(1, D), _w_spec(1, D),
      ],
      out_specs=_row_spec(D, False),
      out_shape=jax.ShapeDtypeStruct((2, N, D), jnp.float32),
      name="tc_layer",
  )(sums, hs, cnt, wl, wr, b, g, bb)


def _tc_layer_head(sums, cnt, hs, wl, wr, b, g, bb, wh, bh):
  return pl.pallas_call(
      _tc_layer_head_body,
      grid=(2, N // ROW_BLOCK),
      in_specs=[
          _sums_spec(),
          _row_spec(D, False),
          _row_spec(LANES, True),
          _w_spec(D, D), _w_spec(D, D), _w_spec(1, D),
          _w_spec(1, D), _w_spec(1, D),
          _w_spec(D, DOUT), _w_spec(1, DOUT),
      ],
      out_specs=_row_spec(DOUT, False),
      out_shape=jax.ShapeDtypeStruct((2, N, DOUT), jnp.float32),
      name="tc_layer_head",
  )(sums, hs, cnt, wl, wr, b, g, bb, wh, bh)


def _pad_edges(edge_index):
  src = edge_index[0]
  dst = edge_index[1]
  pad = EDGES_PER_TYPE - E
  src = jnp.concatenate([src, jnp.zeros((pad,), jnp.int32)])
  dst = jnp.concatenate([dst, jnp.full((pad,), N, jnp.int32)])
  return jnp.stack([src.reshape(N_CHUNKS, CHUNK), dst.reshape(N_CHUNKS, CHUNK)])


def _slabs(hs):
  return hs.reshape(2, N, NUM_SC, DH).transpose(2, 0, 1, 3)


def _stack_layer_params(lp):
  wl = jnp.stack([lp['i2u']['Wl'], lp['u2i']['Wl']])
  wr = jnp.stack([lp['i2u']['Wr'], lp['u2i']['Wr']])
  b = jnp.stack([lp['i2u']['b'], lp['u2i']['b']])[:, None, :]
  g = jnp.stack([lp['ln_u']['g'], lp['ln_i']['g']])[:, None, :]
  bb = jnp.stack([lp['ln_u']['b'], lp['ln_i']['b']])[:, None, :]
  return wl, wr, b, g, bb


@jax.jit
def kernel(x_user, x_item, edge_index_u2i, edge_index_i2u, params):
  edges = jnp.stack([_pad_edges(edge_index_u2i), _pad_edges(edge_index_i2u)])
  hs = jnp.stack([x_user, x_item])

  lp0, lp1 = params['layers']

  sums, cnt = _sc_segsum_builder(True)(_slabs(hs), edges)
  hs = _tc_layer(sums, cnt, hs, *_stack_layer_params(lp0))

  (sums,) = _sc_segsum_builder(False)(_slabs(hs), edges)
  wh = jnp.stack([params['head']['u']['W'], params['head']['i']['W']])
  bh = jnp.stack([params['head']['u']['b'], params['head']['i']['b']])[:, None, :]
  wl, wr, b, g, bb = _stack_layer_params(lp1)
  out = _tc_layer_head(sums, cnt, hs, wl, wr, b, g, bb, wh, bh)

  return out[0], out[1]
